# baseline (speedup 1.0000x reference)
import functools

import jax
import jax.numpy as jnp
from jax import lax
from jax.experimental import pallas as pl
from jax.experimental.pallas import tpu as pltpu
from jax.experimental.pallas import tpu_sc as plsc

B = 4096
V = 100000
D = 16
NF = 26
ND = 13
L = 50

NC = 2
NSUB = 16
NW = NC * NSUB

NROW = (NF + 1) * D
SP_ROWS = NF * D
SP_PER_W = SP_ROWS // NW
BH = B // 2


def _sc_body(tab, sp_idx, sq_idx, out_sp, out_sq,
             rowbuf, spbuf, sqbuf_a, sqbuf_b, outbuf, accbuf, sem):
    wid = lax.axis_index("s") * NC + lax.axis_index("c")

    def stage_row(r):
        pltpu.sync_copy(tab.at[r], rowbuf)

    def sp_unit(k, carry):
        r = wid * SP_PER_W + k
        t = r // D
        stage_row(r)
        pltpu.sync_copy(sp_idx.at[t], spbuf)

        def gather_grp(g, c):
            for j in range(16):
                p = (g * 16 + j) * 16
                outbuf[pl.ds(p, 16)] = plsc.load_gather(
                    rowbuf, [spbuf[pl.ds(p, 16)]])
            return c
        lax.fori_loop(0, B // 256, gather_grp, 0)
        pltpu.sync_copy(outbuf, out_sp.at[r])
        return carry
    lax.fori_loop(0, SP_PER_W, sp_unit, 0)

    d = wid % D
    half = wid // D
    stage_row(SP_ROWS + d)
    col0 = half * BH
    nchunk = BH // 128
    pltpu.async_copy(sq_idx.at[:, pl.ds(col0, 128)], sqbuf_a, sem).wait()

    def seq_compute(buf, col):
        for j in range(8):
            p = j * 16
            accs = [plsc.load_gather(rowbuf, [buf[l, pl.ds(p, 16)]])
                    for l in range(4)]
            for l in range(4, L):
                accs[l % 4] = accs[l % 4] + plsc.load_gather(
                    rowbuf, [buf[l, pl.ds(p, 16)]])
            accbuf[pl.ds(p, 16)] = (accs[0] + accs[1]) + (accs[2] + accs[3])
        pltpu.sync_copy(accbuf, out_sq.at[d, pl.ds(col, 128)])

    def seq_pair(k, carry):
        col = col0 + k * 256
        nxt_b = pltpu.async_copy(
            sq_idx.at[:, pl.ds(col + 128, 128)], sqbuf_b, sem)
        seq_compute(sqbuf_a, col)
        nxt_b.wait()
        col_a = jnp.minimum(col + 256, col0 + (nchunk - 2) * 128)
        nxt_a = pltpu.async_copy(sq_idx.at[:, pl.ds(col_a, 128)], sqbuf_a, sem)
        seq_compute(sqbuf_b, col + 128)
        nxt_a.wait()
        return carry
    lax.fori_loop(0, nchunk // 2 - 1, seq_pair, 0)

    col = col0 + (nchunk - 2) * 128
    nxt_b = pltpu.async_copy(sq_idx.at[:, pl.ds(col + 128, 128)], sqbuf_b, sem)
    seq_compute(sqbuf_a, col)
    nxt_b.wait()
    seq_compute(sqbuf_b, col + 128)


@functools.lru_cache(maxsize=1)
def _sc_gather():
    mesh = plsc.VectorSubcoreMesh(
        core_axis_name="c", subcore_axis_name="s",
        num_cores=NC, num_subcores=NSUB)
    return pl.kernel(
        _sc_body,
        out_type=(
            jax.ShapeDtypeStruct((SP_ROWS, B), jnp.float32),
            jax.ShapeDtypeStruct((D, B), jnp.float32),
        ),
        mesh=mesh,
        compiler_params=pltpu.CompilerParams(
            use_tc_tiling_on_sc=True, needs_layout_passes=False),
        scratch_types=[
            pltpu.VMEM((V,), jnp.float32),
            pltpu.VMEM((B,), jnp.int32),
            pltpu.VMEM((L, 128), jnp.int32),
            pltpu.VMEM((L, 128), jnp.int32),
            pltpu.VMEM((B,), jnp.float32),
            pltpu.VMEM((128,), jnp.float32),
            pltpu.SemaphoreType.DMA,
        ],
    )


def _tc_body(sf, psum, dense, seqf, row0,
             w1, b1, w2, b2, wd, bd, ww, bw, out):
    x_sf = jnp.transpose(sf[...])
    d = dense[...]
    s = seqf[...]

    n0 = jnp.sum((s == 0.0).astype(jnp.float32), axis=1, keepdims=True)
    pooled_sum = jnp.transpose(psum[...]) - n0 * row0[...]
    length = jnp.float32(L) - n0
    pooled = pooled_sum / (length + 1e-8)

    dnn_in = jnp.concatenate([x_sf, pooled, d], axis=1)
    h1 = jnp.maximum(
        jnp.dot(dnn_in, w1[...], preferred_element_type=jnp.float32)
        + b1[...], 0.0)
    h2 = jnp.maximum(
        jnp.dot(h1, w2[...], preferred_element_type=jnp.float32)
        + b2[...], 0.0)
    deep = jnp.dot(h2, wd[...], preferred_element_type=jnp.float32) + bd[...]
    wide = jnp.dot(d, ww[...], preferred_element_type=jnp.float32) + bw[...]
    z = wide + deep
    out[...] = 1.0 / (1.0 + jnp.exp(-z))


def _tc_call(sf, psum, dense, seqf, row0,
             w1, b1, w2, b2, wd, bd, ww, bw):
    BM = 512
    grid = (B // BM,)

    def row_spec(cols):
        return pl.BlockSpec((BM, cols), lambda i: (i, 0))

    def full_spec(r, c):
        return pl.BlockSpec((r, c), lambda i: (0, 0))

    return pl.pallas_call(
        _tc_body,
        grid=grid,
        in_specs=[
            pl.BlockSpec((NF * D, BM), lambda i: (0, i)),
            pl.BlockSpec((D, BM), lambda i: (0, i)),
            row_spec(ND),
            row_spec(L),
            full_spec(1, D),
            full_spec(NF * D + D + ND, 256),
            full_spec(1, 256),
            full_spec(256, 128),
            full_spec(1, 128),
            full_spec(128, 1),
            full_spec(1, 1),
            full_spec(ND, 1),
            full_spec(1, 1),
        ],
        out_specs=pl.BlockSpec((BM, 1), lambda i: (i, 0)),
        out_shape=jax.ShapeDtypeStruct((B, 1), jnp.float32),
    )(sf, psum, dense, seqf, row0,
      w1, b1, w2, b2, wd, bd, ww, bw)


def kernel(inputs, emb_tables, W1, b1, W2, b2, Wd, bd, Ww, bw):
    tab = jnp.swapaxes(emb_tables, 1, 2).reshape(NROW, V)
    sp_idx = inputs[:, :NF].astype(jnp.int32).T
    dense = inputs[:, NF:NF + ND]
    seqf = inputs[:, NF + ND:]
    sq_idx = seqf.astype(jnp.int32).T

    out_sp, out_sq = _sc_gather()(tab, sp_idx, sq_idx)

    row0 = emb_tables[NF, 0:1, :]

    return _tc_call(
        out_sp, out_sq, dense, seqf, row0,
        W1, b1.reshape(1, 256), W2, b2.reshape(1, 128),
        Wd, bd.reshape(1, 1), Ww, bw.reshape(1, 1))


# speedup vs baseline: 1.0046x; 1.0046x over previous
import functools

import jax
import jax.numpy as jnp
from jax import lax
from jax.experimental import pallas as pl
from jax.experimental.pallas import tpu as pltpu
from jax.experimental.pallas import tpu_sc as plsc

B = 4096
V = 100000
D = 16
NF = 26
ND = 13
L = 50

NC = 2
NSUB = 16
NW = NC * NSUB

NROW = (NF + 1) * D
SP_ROWS = NF * D
SP_PER_W = SP_ROWS // NW
BH = B // 2


def _sc_body(tab, sp_idx, sq_idx, out_sp, out_sq,
             rowbuf, spbuf, sqbuf_a, sqbuf_b, outbuf, accbuf, sem):
    wid = lax.axis_index("s") * NC + lax.axis_index("c")

    def stage_row(r):
        pltpu.sync_copy(tab.at[r], rowbuf)

    def sp_unit(k, carry):
        r = wid * SP_PER_W + k
        t = r // D
        stage_row(r)
        pltpu.sync_copy(sp_idx.at[t], spbuf)

        def gather_grp(g, c):
            for j in range(16):
                p = (g * 16 + j) * 16
                outbuf[pl.ds(p, 16)] = plsc.load_gather(
                    rowbuf, [spbuf[pl.ds(p, 16)]])
            return c
        lax.fori_loop(0, B // 256, gather_grp, 0)
        pltpu.sync_copy(outbuf, out_sp.at[r])
        return carry
    lax.fori_loop(0, SP_PER_W, sp_unit, 0)

    d = wid % D
    half = wid // D
    stage_row(SP_ROWS + d)
    col0 = half * BH
    nchunk = BH // 128
    pltpu.async_copy(sq_idx.at[:, pl.ds(col0, 128)], sqbuf_a, sem).wait()

    def seq_compute(buf, col):
        for j in range(8):
            p = j * 16
            accs = [plsc.load_gather(rowbuf, [buf[l, pl.ds(p, 16)]])
                    for l in range(4)]
            for l in range(4, L):
                accs[l % 4] = accs[l % 4] + plsc.load_gather(
                    rowbuf, [buf[l, pl.ds(p, 16)]])
            accbuf[pl.ds(p, 16)] = (accs[0] + accs[1]) + (accs[2] + accs[3])
        pltpu.sync_copy(accbuf, out_sq.at[d, pl.ds(col, 128)])

    def seq_pair(k, carry):
        col = col0 + k * 256
        nxt_b = pltpu.async_copy(
            sq_idx.at[:, pl.ds(col + 128, 128)], sqbuf_b, sem)
        seq_compute(sqbuf_a, col)
        nxt_b.wait()
        col_a = jnp.minimum(col + 256, col0 + (nchunk - 2) * 128)
        nxt_a = pltpu.async_copy(sq_idx.at[:, pl.ds(col_a, 128)], sqbuf_a, sem)
        seq_compute(sqbuf_b, col + 128)
        nxt_a.wait()
        return carry
    lax.fori_loop(0, nchunk // 2 - 1, seq_pair, 0)

    col = col0 + (nchunk - 2) * 128
    nxt_b = pltpu.async_copy(sq_idx.at[:, pl.ds(col + 128, 128)], sqbuf_b, sem)
    seq_compute(sqbuf_a, col)
    nxt_b.wait()
    seq_compute(sqbuf_b, col + 128)


@functools.lru_cache(maxsize=1)
def _sc_gather():
    mesh = plsc.VectorSubcoreMesh(
        core_axis_name="c", subcore_axis_name="s",
        num_cores=NC, num_subcores=NSUB)
    return pl.kernel(
        _sc_body,
        out_type=(
            jax.ShapeDtypeStruct((SP_ROWS, B), jnp.float32),
            jax.ShapeDtypeStruct((D, B), jnp.float32),
        ),
        mesh=mesh,
        compiler_params=pltpu.CompilerParams(
            use_tc_tiling_on_sc=True, needs_layout_passes=False),
        scratch_types=[
            pltpu.VMEM((V,), jnp.float32),
            pltpu.VMEM((B,), jnp.int32),
            pltpu.VMEM((L, 128), jnp.int32),
            pltpu.VMEM((L, 128), jnp.int32),
            pltpu.VMEM((B,), jnp.float32),
            pltpu.VMEM((128,), jnp.float32),
            pltpu.SemaphoreType.DMA,
        ],
    )


def _tc_body(sf, psum, dense, seqf, row0,
             w1, b1, w2, b2, wd, bd, ww, bw, out):
    x_sf = jnp.transpose(sf[...])
    d = dense[...]
    s = seqf[...]

    n0 = jnp.sum((s == 0.0).astype(jnp.float32), axis=1, keepdims=True)
    pooled_sum = jnp.transpose(psum[...]) - n0 * row0[...]
    length = jnp.float32(L) - n0
    pooled = pooled_sum / (length + 1e-8)

    dnn_in = jnp.concatenate([x_sf, pooled, d], axis=1)
    h1 = jnp.maximum(
        jnp.dot(dnn_in, w1[...], preferred_element_type=jnp.float32)
        + b1[...], 0.0)
    h2 = jnp.maximum(
        jnp.dot(h1, w2[...], preferred_element_type=jnp.float32)
        + b2[...], 0.0)
    deep = jnp.dot(h2, wd[...], preferred_element_type=jnp.float32) + bd[...]
    wide = jnp.dot(d, ww[...], preferred_element_type=jnp.float32) + bw[...]
    z = wide + deep
    out[...] = 1.0 / (1.0 + jnp.exp(-z))


def _tc_call(sf, psum, dense, seqf, row0,
             w1, b1, w2, b2, wd, bd, ww, bw):
    BM = 1024
    grid = (B // BM,)

    def row_spec(cols):
        return pl.BlockSpec((BM, cols), lambda i: (i, 0))

    def full_spec(r, c):
        return pl.BlockSpec((r, c), lambda i: (0, 0))

    return pl.pallas_call(
        _tc_body,
        grid=grid,
        in_specs=[
            pl.BlockSpec((NF * D, BM), lambda i: (0, i)),
            pl.BlockSpec((D, BM), lambda i: (0, i)),
            row_spec(ND),
            row_spec(L),
            full_spec(1, D),
            full_spec(NF * D + D + ND, 256),
            full_spec(1, 256),
            full_spec(256, 128),
            full_spec(1, 128),
            full_spec(128, 1),
            full_spec(1, 1),
            full_spec(ND, 1),
            full_spec(1, 1),
        ],
        out_specs=pl.BlockSpec((BM, 1), lambda i: (i, 0)),
        out_shape=jax.ShapeDtypeStruct((B, 1), jnp.float32),
    )(sf, psum, dense, seqf, row0,
      w1, b1, w2, b2, wd, bd, ww, bw)


def kernel(inputs, emb_tables, W1, b1, W2, b2, Wd, bd, Ww, bw):
    tab = jnp.swapaxes(emb_tables, 1, 2).reshape(NROW, V)
    sp_idx = inputs[:, :NF].astype(jnp.int32).T
    dense = inputs[:, NF:NF + ND]
    seqf = inputs[:, NF + ND:]
    sq_idx = seqf.astype(jnp.int32).T

    out_sp, out_sq = _sc_gather()(tab, sp_idx, sq_idx)

    row0 = emb_tables[NF, 0:1, :]

    return _tc_call(
        out_sp, out_sq, dense, seqf, row0,
        W1, b1.reshape(1, 256), W2, b2.reshape(1, 128),
        Wd, bd.reshape(1, 1), Ww, bw.reshape(1, 1))
